# baseline (speedup 1.0000x reference)
import functools

import jax
import jax.numpy as jnp
from jax import lax
from jax.experimental import pallas as pl
from jax.experimental.pallas import tpu as pltpu
from jax.experimental.pallas import tpu_sc as plsc

VOCAB = 100000
DIM = 64
B, H, W, L = 1024, 4, 4, 20
CELLS = B * H * W
NW = 32
CELLS_PER_W = CELLS // NW
IDX_PER_W = CELLS_PER_W * L
ROWS_BLK = 5000


def _matvec_body(t_ref, w_ref, b_ref, o_ref):
    prod = t_ref[...] * w_ref[...]
    s = jnp.sum(prod, axis=1, keepdims=True)
    o_ref[...] = s + b_ref[0, 0]


def _project_table(table, pred_w, pred_b):
    pred_w = pred_w.astype(jnp.float32)
    b20 = (pred_b.astype(jnp.float32) / jnp.float32(L)).reshape(1, 1)
    p2 = pl.pallas_call(
        _matvec_body,
        grid=(VOCAB // ROWS_BLK,),
        in_specs=[
            pl.BlockSpec((ROWS_BLK, DIM), lambda i: (i, jnp.int32(0))),
            pl.BlockSpec((1, DIM), lambda i: (jnp.int32(0), jnp.int32(0))),
            pl.BlockSpec((1, 1), lambda i: (jnp.int32(0), jnp.int32(0))),
        ],
        out_specs=pl.BlockSpec((ROWS_BLK, 1), lambda i: (i, jnp.int32(0))),
        out_shape=jax.ShapeDtypeStruct((VOCAB, 1), jnp.float32),
    )(table, pred_w, b20)
    return p2.reshape(VOCAB)


@functools.lru_cache(maxsize=1)
def _make_sc_gather_sum():
    mesh = plsc.VectorSubcoreMesh(core_axis_name="c", subcore_axis_name="s")

    @functools.partial(
        pl.kernel,
        mesh=mesh,
        out_type=jax.ShapeDtypeStruct((CELLS,), jnp.float32),
        scratch_types=[
            pltpu.VMEM((VOCAB,), jnp.float32),
            pltpu.VMEM((IDX_PER_W,), jnp.int32),
            pltpu.VMEM((CELLS_PER_W,), jnp.float32),
        ],
        compiler_params=pltpu.CompilerParams(needs_layout_passes=False),
    )
    def _sc_gather_sum(p_hbm, idx_hbm, out_hbm, p_v, idx_v, acc_v):
        wid = lax.axis_index("s") * 2 + lax.axis_index("c")
        pltpu.sync_copy(p_hbm, p_v)
        pltpu.sync_copy(idx_hbm.at[pl.ds(wid * IDX_PER_W, IDX_PER_W)], idx_v)
        iota20 = lax.iota(jnp.int32, 16) * jnp.int32(L)

        def body(c, carry):
            base = c * jnp.int32(16 * L)
            acc = jnp.zeros((16,), jnp.float32)
            for l in range(L):
                pos = iota20 + (base + jnp.int32(l))
                iv = plsc.load_gather(idx_v, [pos])
                acc = acc + plsc.load_gather(p_v, [iv])
            acc_v[pl.ds(c * jnp.int32(16), 16)] = acc
            return carry

        lax.fori_loop(
            jnp.int32(0), jnp.int32(CELLS_PER_W // 16), body, jnp.int32(0)
        )
        pltpu.sync_copy(acc_v, out_hbm.at[pl.ds(wid * CELLS_PER_W, CELLS_PER_W)])

    return _sc_gather_sum


def kernel(x, table, pred_w, pred_b):
    p = _project_table(table, pred_w, pred_b)
    xi = x.astype(jnp.int32).reshape(CELLS * L)
    out_flat = _make_sc_gather_sum()(p, xi)
    return out_flat.reshape(B, H, W).astype(jnp.float64)


# speedup vs baseline: 1.8859x; 1.8859x over previous
import functools

import jax
import jax.numpy as jnp
from jax import lax
from jax.experimental import pallas as pl
from jax.experimental.pallas import tpu as pltpu
from jax.experimental.pallas import tpu_sc as plsc

VOCAB = 100000
DIM = 64
B, H, W, L = 1024, 4, 4, 20
CELLS = B * H * W
NW = 32
CELLS_PER_W = CELLS // NW
IDX_PER_W = CELLS_PER_W * L
COLS_BLK = 12800


def _matvec_body(w_ref, t_ref, b_ref, o_ref):
    o_ref[...] = (
        jnp.dot(w_ref[...], t_ref[...], preferred_element_type=jnp.float32,
                precision=jax.lax.Precision.HIGHEST)
        + b_ref[0, 0]
    )


def _project_table(table, pred_w, pred_b):
    tt = table.T
    pred_w = pred_w.astype(jnp.float32)
    b20 = (pred_b.astype(jnp.float32) / jnp.float32(L)).reshape(1, 1)
    grid = (VOCAB + COLS_BLK - 1) // COLS_BLK
    p2 = pl.pallas_call(
        _matvec_body,
        grid=(grid,),
        in_specs=[
            pl.BlockSpec((1, DIM), lambda i: (jnp.int32(0), jnp.int32(0))),
            pl.BlockSpec((DIM, COLS_BLK), lambda i: (jnp.int32(0), i)),
            pl.BlockSpec((1, 1), lambda i: (jnp.int32(0), jnp.int32(0))),
        ],
        out_specs=pl.BlockSpec((1, COLS_BLK), lambda i: (jnp.int32(0), i)),
        out_shape=jax.ShapeDtypeStruct((1, VOCAB), jnp.float32),
    )(pred_w, tt, b20)
    return p2.reshape(VOCAB)


@functools.lru_cache(maxsize=1)
def _make_sc_gather_sum():
    mesh = plsc.VectorSubcoreMesh(core_axis_name="c", subcore_axis_name="s")

    @functools.partial(
        pl.kernel,
        mesh=mesh,
        out_type=jax.ShapeDtypeStruct((CELLS,), jnp.float32),
        scratch_types=[
            pltpu.VMEM((VOCAB,), jnp.float32),
            pltpu.VMEM((IDX_PER_W,), jnp.int32),
            pltpu.VMEM((CELLS_PER_W,), jnp.float32),
        ],
        compiler_params=pltpu.CompilerParams(needs_layout_passes=False),
    )
    def _sc_gather_sum(p_hbm, idx_hbm, out_hbm, p_v, idx_v, acc_v):
        wid = lax.axis_index("s") * 2 + lax.axis_index("c")
        pltpu.sync_copy(p_hbm, p_v)
        pltpu.sync_copy(idx_hbm.at[pl.ds(wid * IDX_PER_W, IDX_PER_W)], idx_v)
        iota20 = lax.iota(jnp.int32, 16) * jnp.int32(L)

        def body(c, carry):
            base = c * jnp.int32(16 * L)
            acc = jnp.zeros((16,), jnp.float32)
            for l in range(L):
                pos = iota20 + (base + jnp.int32(l))
                iv = plsc.load_gather(idx_v, [pos])
                acc = acc + plsc.load_gather(p_v, [iv])
            acc_v[pl.ds(c * jnp.int32(16), 16)] = acc
            return carry

        lax.fori_loop(
            jnp.int32(0), jnp.int32(CELLS_PER_W // 16), body, jnp.int32(0)
        )
        pltpu.sync_copy(acc_v, out_hbm.at[pl.ds(wid * CELLS_PER_W, CELLS_PER_W)])

    return _sc_gather_sum


def kernel(x, table, pred_w, pred_b):
    p = _project_table(table, pred_w, pred_b)
    xi = x.astype(jnp.int32).reshape(CELLS * L)
    out_flat = _make_sc_gather_sum()(p, xi)
    return out_flat.reshape(B, H, W).astype(jnp.float64)
